# baseline (speedup 1.0000x reference)
import functools

import jax
import jax.numpy as jnp
from jax import lax
from jax.experimental import pallas as pl
from jax.experimental.pallas import tpu as pltpu
from jax.experimental.pallas import tpu_sc as plsc

_D = 64
_NW = 32
_V = 100000
_RW = 3200
_RCH = 400
_XR_W = 128
_XCH = 4
_NCH_G = _XR_W // _XCH


def _mesh():
    return plsc.VectorSubcoreMesh(core_axis_name="c", subcore_axis_name="s")


@jax.jit
def _widen(weight):
    @functools.partial(
        pl.kernel,
        out_type=jax.ShapeDtypeStruct((_V, 128), jnp.float32),
        mesh=_mesh(),
        scratch_types=[
            pltpu.VMEM((_RCH, _D), jnp.float32),
            pltpu.VMEM((_RCH, 128), jnp.float32),
        ],
        compiler_params=pltpu.CompilerParams(use_tc_tiling_on_sc=True),
    )
    def conv(w_hbm, w2_hbm, a_v, b_v):
        wid = lax.axis_index("s") * 2 + lax.axis_index("c")
        r0 = jnp.minimum(wid * _RW, _V - _RW)

        def chunk(c, _):
            pltpu.sync_copy(w_hbm.at[pl.ds(r0 + _RCH * c, _RCH)], a_v)

            def row(q, _):
                for k0 in range(0, _D, 16):
                    b_v[q, pl.ds(k0, 16)] = a_v[q, pl.ds(k0, 16)]
                return 0

            lax.fori_loop(0, _RCH, row, 0)
            pltpu.sync_copy(b_v, w2_hbm.at[pl.ds(r0 + _RCH * c, _RCH)])
            return 0

        lax.fori_loop(0, _RW // _RCH, chunk, 0)

    return conv(weight)


@functools.partial(jax.jit, static_argnums=(2, 3))
def _gather(x32, w2, b, s):
    @functools.partial(
        pl.kernel,
        out_type=jax.ShapeDtypeStruct((b, s, _D), jnp.float32),
        mesh=_mesh(),
        scratch_types=[
            pltpu.VMEM((_XR_W, s), jnp.int32),
            pltpu.VMEM((2, _XCH, s, 128), jnp.float32),
            pltpu.VMEM((2, _XCH, s, _D), jnp.float32),
            pltpu.SemaphoreType.DMA((2,)),
            pltpu.SemaphoreType.DMA((2,)),
        ],
        compiler_params=pltpu.CompilerParams(use_tc_tiling_on_sc=True),
    )
    def gath(w2_hbm, x_hbm, out_hbm, idx_v, pair_v, stage_v, gsem, ssem):
        wid = lax.axis_index("s") * 2 + lax.axis_index("c")
        bx = wid * _XR_W
        pltpu.sync_copy(x_hbm.at[pl.ds(bx, _XR_W)], idx_v)

        def fire_g(c, slot):
            for j in range(_XCH):
                pltpu.async_copy(
                    w2_hbm.at[idx_v.at[_XCH * c + j]],
                    pair_v.at[slot].at[j],
                    gsem.at[slot],
                )

        def drain_g(c, slot):
            for j in range(_XCH):
                pltpu.make_async_copy(
                    w2_hbm.at[idx_v.at[_XCH * c + j]],
                    pair_v.at[slot].at[j],
                    gsem.at[slot],
                ).wait()

        def extract(c, slot):
            def ej(j, _):
                def et(t, _):
                    for k0 in range(0, _D, 16):
                        stage_v[slot, j, t, pl.ds(k0, 16)] = pair_v[
                            slot, j, t, pl.ds(k0, 16)
                        ]
                    return 0

                lax.fori_loop(0, s, et, 0)
                return 0

            lax.fori_loop(0, _XCH, ej, 0)

        def fire_w(c, slot):
            pltpu.async_copy(
                stage_v.at[slot],
                out_hbm.at[pl.ds(bx + _XCH * c, _XCH)],
                ssem.at[slot],
            )

        def drain_w(c, slot):
            pltpu.make_async_copy(
                stage_v.at[slot],
                out_hbm.at[pl.ds(bx + _XCH * c, _XCH)],
                ssem.at[slot],
            ).wait()

        fire_g(0, 0)
        fire_g(1, 1)
        for c in (0, 1):
            drain_g(c, c)
            extract(c, c)
            fire_w(c, c)
            fire_g(c + 2, c)

        def body(i, _):
            for sl in (0, 1):
                c = 2 * i + sl
                drain_g(c, sl)
                drain_w(c - 2, sl)
                extract(c, sl)
                fire_w(c, sl)
                fire_g(c + 2, sl)
            return 0

        lax.fori_loop(1, _NCH_G // 2 - 1, body, 0)

        for c in (_NCH_G - 2, _NCH_G - 1):
            sl = c % 2
            drain_g(c, sl)
            drain_w(c - 2, sl)
            extract(c, sl)
            fire_w(c, sl)
        drain_w(_NCH_G - 2, 0)
        drain_w(_NCH_G - 1, 1)

    return gath(w2, x32)


def kernel(x, weight):
    b, s = x.shape
    w2 = _widen(weight)
    return _gather(x.astype(jnp.int32), w2, b, s)
